# Initial kernel scaffold; baseline (speedup 1.0000x reference)
import functools

import jax
import jax.numpy as jnp
from jax import lax
from jax.experimental import pallas as pl
from jax.experimental.pallas import tpu as pltpu
from jax.experimental.pallas import tpu_sc as plsc

COULOMB_CONSTANT = 14.399645478425668
CUTOFF = 10.0
R_ON = 0.8 * CUTOFF
INV_W = 1.0 / (CUTOFF - R_ON)
PI = 3.14159265358979323846

S1 = 1.0
S3 = -1.0 / 6.0
S5 = 1.0 / 120.0
S7 = -1.0 / 5040.0
S9 = 1.0 / 362880.0

NC = 2
NS = 16
NW = NC * NS
L = 16
CH = 2000


def _sc_coulomb(n_pad, e_pad):
    epw = e_pad // NW
    nchunk = epw // CH
    sl = n_pad // NS

    mesh = plsc.VectorSubcoreMesh(core_axis_name="c", subcore_axis_name="s")

    @functools.partial(
        pl.kernel,
        out_type=jax.ShapeDtypeStruct((NC, n_pad), jnp.float32),
        mesh=mesh,
        scratch_types=[
            pltpu.VMEM((n_pad,), jnp.float32),
            pltpu.VMEM((n_pad,), jnp.float32),
            pltpu.VMEM((CH,), jnp.int32),
            pltpu.VMEM((CH,), jnp.int32),
            pltpu.VMEM((CH,), jnp.float32),
            pltpu.VMEM((n_pad // NS,), jnp.float32),
            pltpu.VMEM((n_pad // NS,), jnp.float32),
            pltpu.VMEM_SHARED((NS, n_pad), jnp.float32),
        ],
    )
    def kern(center_hbm, neighbor_hbm, length_hbm, charges_hbm, out_hbm,
             table, acc, cbuf, nbuf, lbuf, red0, red1, shared):
        c = lax.axis_index("c")
        s = lax.axis_index("s")
        wid = c * NS + s

        pltpu.sync_copy(charges_hbm, table)

        zero16 = jnp.zeros((L,), jnp.float32)

        @pl.loop(0, n_pad, step=L)
        def _(i):
            acc[pl.ds(i, L)] = zero16

        base_w = wid * epw

        @pl.loop(0, nchunk)
        def _(j):
            base = base_w + j * CH
            pltpu.sync_copy(center_hbm.at[pl.ds(base, CH)], cbuf)
            pltpu.sync_copy(neighbor_hbm.at[pl.ds(base, CH)], nbuf)
            pltpu.sync_copy(length_hbm.at[pl.ds(base, CH)], lbuf)

            @pl.loop(0, CH, step=L)
            def _(i):
                cidx = cbuf[pl.ds(i, L)]
                nidx = nbuf[pl.ds(i, L)]
                d = lbuf[pl.ds(i, L)]
                q1 = plsc.load_gather(table, [cidx])
                q2 = plsc.load_gather(table, [nidx])
                t = jnp.clip((d - R_ON) * INV_W, 0.0, 1.0)
                x = (t - 0.5) * PI
                x2 = x * x
                sinx = x * (S1 + x2 * (S3 + x2 * (S5 + x2 * (S7 + x2 * S9))))
                pair = (0.25 * COULOMB_CONSTANT) * (1.0 - sinx) * q1 * q2 / d
                plsc.addupdate_scatter(acc, [cidx], pair)

        pltpu.sync_copy(acc, shared.at[s])
        plsc.subcore_barrier()

        off = s * sl
        pltpu.sync_copy(shared.at[0, pl.ds(off, sl)], red0)
        for t_ in range(1, NS):
            pltpu.sync_copy(shared.at[t_, pl.ds(off, sl)], red1)

            @pl.loop(0, sl, step=L)
            def _(i):
                red0[pl.ds(i, L)] = red0[pl.ds(i, L)] + red1[pl.ds(i, L)]

        pltpu.sync_copy(red0, out_hbm.at[c, pl.ds(off, sl)])

    return kern


def _tc_sum(partials):
    n_pad = partials.shape[1]

    def body(p_ref, o_ref):
        o_ref[...] = p_ref[0:1, :] + p_ref[1:2, :]

    return pl.pallas_call(
        body,
        out_shape=jax.ShapeDtypeStruct((1, n_pad), jnp.float32),
    )(partials)


def kernel(long_edge_index, long_edge_length, atomic_charges):
    n = atomic_charges.shape[0]
    e = long_edge_length.shape[0]

    n_pad = ((n + 1 + 255) // 256) * 256
    epb = NW * CH
    e_pad = ((e + epb - 1) // epb) * epb

    center = long_edge_index[0].astype(jnp.int32)
    neighbor = long_edge_index[1].astype(jnp.int32)
    length = long_edge_length.astype(jnp.float32)
    if e_pad != e:
        center = jnp.pad(center, (0, e_pad - e), constant_values=n)
        neighbor = jnp.pad(neighbor, (0, e_pad - e), constant_values=n)
        length = jnp.pad(length, (0, e_pad - e), constant_values=1.0)
    charges = jnp.pad(atomic_charges.astype(jnp.float32), (0, n_pad - n))

    partials = _sc_coulomb(n_pad, e_pad)(center, neighbor, length, charges)
    summed = _tc_sum(partials)
    return summed[0, :n]


# speedup vs baseline: 99.2885x; 99.2885x over previous
import functools

import jax
import jax.numpy as jnp
from jax import lax
from jax.experimental import pallas as pl
from jax.experimental.pallas import tpu as pltpu
from jax.experimental.pallas import tpu_sc as plsc

COULOMB_CONSTANT = 14.399645478425668
CUTOFF = 10.0
R_ON = 0.8 * CUTOFF
INV_W = 1.0 / (CUTOFF - R_ON)
PI = 3.14159265358979323846

S1 = 1.0
S3 = -1.0 / 6.0
S5 = 1.0 / 120.0
S7 = -1.0 / 5040.0
S9 = 1.0 / 362880.0

NC = 2
NS = 16
NW = NC * NS
L = 16
CH = 2000


def _sc_coulomb(n_pad, e_pad):
    epw = e_pad // NW
    nchunk = epw // CH

    mesh = plsc.VectorSubcoreMesh(core_axis_name="c", subcore_axis_name="s")

    @functools.partial(
        pl.kernel,
        out_type=jax.ShapeDtypeStruct((NW * n_pad,), jnp.float32),
        mesh=mesh,
        compiler_params=pltpu.CompilerParams(needs_layout_passes=False),
        scratch_types=[
            pltpu.VMEM((n_pad,), jnp.float32),
            pltpu.VMEM((n_pad,), jnp.float32),
            pltpu.VMEM((CH,), jnp.int32),
            pltpu.VMEM((CH,), jnp.int32),
            pltpu.VMEM((CH,), jnp.float32),
        ],
    )
    def kern(center_hbm, neighbor_hbm, length_hbm, charges_hbm, out_hbm,
             table, acc, cbuf, nbuf, lbuf):
        c = lax.axis_index("c")
        s = lax.axis_index("s")
        wid = c * NS + s

        pltpu.sync_copy(charges_hbm, table)

        zero16 = jnp.zeros((L,), jnp.float32)

        @pl.loop(0, n_pad, step=L)
        def _(i):
            acc[pl.ds(i, L)] = zero16

        base_w = wid * epw

        @pl.loop(0, nchunk)
        def _(j):
            base = base_w + j * CH
            pltpu.sync_copy(center_hbm.at[pl.ds(base, CH)], cbuf)
            pltpu.sync_copy(neighbor_hbm.at[pl.ds(base, CH)], nbuf)
            pltpu.sync_copy(length_hbm.at[pl.ds(base, CH)], lbuf)

            @pl.loop(0, CH, step=L)
            def _(i):
                cidx = cbuf[pl.ds(i, L)]
                nidx = nbuf[pl.ds(i, L)]
                d = lbuf[pl.ds(i, L)]
                q1 = plsc.load_gather(table, [cidx])
                q2 = plsc.load_gather(table, [nidx])
                t = jnp.clip((d - R_ON) * INV_W, 0.0, 1.0)
                x = (t - 0.5) * PI
                x2 = x * x
                sinx = x * (S1 + x2 * (S3 + x2 * (S5 + x2 * (S7 + x2 * S9))))
                pair = (0.25 * COULOMB_CONSTANT) * (1.0 - sinx) * q1 * q2 / d
                plsc.addupdate_scatter(acc, [cidx], pair)

        pltpu.sync_copy(acc, out_hbm.at[pl.ds(wid * n_pad, n_pad)])

    return kern


def _tc_sum(partials):
    n_pad = partials.shape[1]

    def body(p_ref, o_ref):
        o_ref[...] = jnp.sum(p_ref[...], axis=0, keepdims=True)

    return pl.pallas_call(
        body,
        out_shape=jax.ShapeDtypeStruct((1, n_pad), jnp.float32),
    )(partials)


def kernel(long_edge_index, long_edge_length, atomic_charges):
    n = atomic_charges.shape[0]
    e = long_edge_length.shape[0]

    n_pad = ((n + 1 + 255) // 256) * 256
    epb = NW * CH
    e_pad = ((e + epb - 1) // epb) * epb

    center = long_edge_index[0].astype(jnp.int32)
    neighbor = long_edge_index[1].astype(jnp.int32)
    length = long_edge_length.astype(jnp.float32)
    if e_pad != e:
        center = jnp.pad(center, (0, e_pad - e), constant_values=n)
        neighbor = jnp.pad(neighbor, (0, e_pad - e), constant_values=n)
        length = jnp.pad(length, (0, e_pad - e), constant_values=1.0)
    charges = jnp.pad(atomic_charges.astype(jnp.float32), (0, n_pad - n))

    partials = _sc_coulomb(n_pad, e_pad)(center, neighbor, length, charges)
    summed = _tc_sum(partials.reshape(NW, n_pad))
    return summed[0, :n]


# speedup vs baseline: 195.2381x; 1.9664x over previous
import functools

import jax
import jax.numpy as jnp
from jax import lax
from jax.experimental import pallas as pl
from jax.experimental.pallas import tpu as pltpu
from jax.experimental.pallas import tpu_sc as plsc

COULOMB_CONSTANT = 14.399645478425668
CUTOFF = 10.0
R_ON = 0.8 * CUTOFF
HALF_PI = 1.5707963267948966
PI = 3.141592653589793
SCALE = PI / (CUTOFF - R_ON)

S3 = -1.0 / 6.0
S5 = 1.0 / 120.0
S7 = -1.0 / 5040.0

NC = 2
NS = 16
NW = NC * NS
L = 16
CH = 2000


def _sc_coulomb(n_pad, e_pad):
    epw = e_pad // NW
    nchunk = epw // CH

    mesh = plsc.VectorSubcoreMesh(core_axis_name="c", subcore_axis_name="s")

    @functools.partial(
        pl.kernel,
        out_type=jax.ShapeDtypeStruct((NW * n_pad,), jnp.float32),
        mesh=mesh,
        compiler_params=pltpu.CompilerParams(needs_layout_passes=False),
        scratch_types=[
            pltpu.VMEM((n_pad,), jnp.float32),
            pltpu.VMEM((n_pad,), jnp.float32),
            pltpu.VMEM((2 * CH,), jnp.int32),
            pltpu.VMEM((2 * CH,), jnp.float32),
            pltpu.SemaphoreType.DMA,
            pltpu.SemaphoreType.DMA,
            pltpu.SemaphoreType.DMA,
        ],
    )
    def kern(packed_hbm, length_hbm, charges_hbm, out_hbm,
             table, acc, ibuf, lbuf, tsem, sem0, sem1):
        c = lax.axis_index("c")
        s = lax.axis_index("s")
        wid = c * NS + s
        base_w = wid * epw

        def issue(j, half, sem):
            base = base_w + j * CH
            off = half * CH
            pltpu.async_copy(packed_hbm.at[pl.ds(base, CH)],
                             ibuf.at[pl.ds(off, CH)], sem)
            pltpu.async_copy(length_hbm.at[pl.ds(base, CH)],
                             lbuf.at[pl.ds(off, CH)], sem)

        def drain(half, sem):
            off = half * CH
            pltpu.make_async_copy(packed_hbm.at[pl.ds(base_w, CH)],
                                  ibuf.at[pl.ds(off, CH)], sem).wait()
            pltpu.make_async_copy(length_hbm.at[pl.ds(base_w, CH)],
                                  lbuf.at[pl.ds(off, CH)], sem).wait()

        def compute(half):
            off = half * CH

            @plsc.parallel_loop(0, CH, L, unroll=4)
            def _(i):
                packed = ibuf[pl.ds(off + i, L)]
                d = lbuf[pl.ds(off + i, L)]
                cidx = lax.bitwise_and(packed, 0xFFFF)
                nidx = lax.shift_right_logical(packed, 16)
                q1 = plsc.load_gather(table, [cidx])
                q2 = plsc.load_gather(table, [nidx])
                x = jnp.clip((d - R_ON) * SCALE, 0.0, PI) - HALF_PI
                x2 = x * x
                sinx = x * (1.0 + x2 * (S3 + x2 * (S5 + x2 * S7)))
                pair = ((0.25 * COULOMB_CONSTANT) * (1.0 - sinx)) * q1 * q2 / d
                plsc.addupdate_scatter(acc, [cidx], pair)

        tcopy = pltpu.async_copy(charges_hbm, table, tsem)
        issue(0, 0, sem0)

        zero16 = jnp.zeros((L,), jnp.float32)

        @plsc.parallel_loop(0, n_pad, L, unroll=8)
        def _(i):
            acc[pl.ds(i, L)] = zero16

        tcopy.wait()

        @pl.loop(0, nchunk, step=2)
        def _(j):
            @pl.when(j + 1 < nchunk)
            def _():
                issue(j + 1, 1, sem1)

            drain(0, sem0)
            compute(0)

            @pl.when(j + 2 < nchunk)
            def _():
                issue(j + 2, 0, sem0)

            @pl.when(j + 1 < nchunk)
            def _():
                drain(1, sem1)
                compute(1)

        pltpu.sync_copy(acc, out_hbm.at[pl.ds(wid * n_pad, n_pad)])

    return kern


def _tc_sum(partials):
    n_pad = partials.shape[1]

    def body(p_ref, o_ref):
        o_ref[...] = jnp.sum(p_ref[...], axis=0, keepdims=True)

    return pl.pallas_call(
        body,
        out_shape=jax.ShapeDtypeStruct((1, n_pad), jnp.float32),
    )(partials)


def kernel(long_edge_index, long_edge_length, atomic_charges):
    n = atomic_charges.shape[0]
    e = long_edge_length.shape[0]

    n_pad = ((n + 1 + 255) // 256) * 256
    if n_pad > 65536:
        raise ValueError("node count too large for u16 index packing")
    epb = NW * CH
    e_pad = ((e + epb - 1) // epb) * epb

    center = long_edge_index[0].astype(jnp.int32)
    neighbor = long_edge_index[1].astype(jnp.int32)
    length = long_edge_length.astype(jnp.float32)
    if e_pad != e:
        center = jnp.pad(center, (0, e_pad - e), constant_values=n)
        neighbor = jnp.pad(neighbor, (0, e_pad - e), constant_values=n)
        length = jnp.pad(length, (0, e_pad - e), constant_values=1.0)
    packed = jnp.bitwise_or(center, jnp.left_shift(neighbor, 16))
    charges = jnp.pad(atomic_charges.astype(jnp.float32), (0, n_pad - n))

    partials = _sc_coulomb(n_pad, e_pad)(packed, length, charges)
    summed = _tc_sum(partials.reshape(NW, n_pad))
    return summed[0, :n]


# speedup vs baseline: 327.8196x; 1.6791x over previous
import functools

import jax
import jax.numpy as jnp
from jax import lax
from jax.experimental import pallas as pl
from jax.experimental.pallas import tpu as pltpu
from jax.experimental.pallas import tpu_sc as plsc

COULOMB_CONSTANT = 14.399645478425668
CUTOFF = 10.0
R_ON = 0.8 * CUTOFF
HALF_PI = 1.5707963267948966
PI = 3.141592653589793
SCALE = PI / (CUTOFF - R_ON)

C0 = 0.25 * COULOMB_CONSTANT
C3 = C0 * (-1.0 / 6.0)
C5 = C0 * (1.0 / 120.0)
C7 = C0 * (-1.0 / 5040.0)

NC = 2
NS = 16
NW = NC * NS
L = 16
CH = 2000


def _sc_coulomb(n, n_pad, e_pad):
    epw = e_pad // NW
    nchunk = epw // CH

    mesh = plsc.VectorSubcoreMesh(core_axis_name="c", subcore_axis_name="s")

    @functools.partial(
        pl.kernel,
        out_type=jax.ShapeDtypeStruct((NW * n_pad,), jnp.float32),
        mesh=mesh,
        compiler_params=pltpu.CompilerParams(needs_layout_passes=False),
        scratch_types=[
            pltpu.VMEM((n_pad,), jnp.float32),
            pltpu.VMEM((n_pad,), jnp.float32),
            pltpu.VMEM((2 * CH,), jnp.int32),
            pltpu.VMEM((2 * CH,), jnp.int32),
            pltpu.VMEM((2 * CH,), jnp.float32),
            pltpu.SemaphoreType.DMA,
            pltpu.SemaphoreType.DMA,
            pltpu.SemaphoreType.DMA,
        ],
    )
    def kern(idx2_hbm, length_hbm, charges_hbm, out_hbm,
             table, acc, cbuf, nbuf, lbuf, tsem, sem0, sem1):
        c = lax.axis_index("c")
        s = lax.axis_index("s")
        wid = c * NS + s
        base_w = wid * epw

        def issue(j, half, sem):
            base = base_w + j * CH
            off = half * CH
            pltpu.async_copy(idx2_hbm.at[pl.ds(base, CH)],
                             cbuf.at[pl.ds(off, CH)], sem)
            pltpu.async_copy(idx2_hbm.at[pl.ds(e_pad + base, CH)],
                             nbuf.at[pl.ds(off, CH)], sem)
            pltpu.async_copy(length_hbm.at[pl.ds(base, CH)],
                             lbuf.at[pl.ds(off, CH)], sem)

        def drain(half, sem):
            off = half * CH
            pltpu.make_async_copy(idx2_hbm.at[pl.ds(base_w, CH)],
                                  cbuf.at[pl.ds(off, CH)], sem).wait()
            pltpu.make_async_copy(idx2_hbm.at[pl.ds(base_w, CH)],
                                  nbuf.at[pl.ds(off, CH)], sem).wait()
            pltpu.make_async_copy(length_hbm.at[pl.ds(base_w, CH)],
                                  lbuf.at[pl.ds(off, CH)], sem).wait()

        def compute(half):
            off = half * CH

            @plsc.parallel_loop(0, CH, L, unroll=4)
            def _(i):
                cidx = cbuf[pl.ds(off + i, L)]
                nidx = nbuf[pl.ds(off + i, L)]
                d = lbuf[pl.ds(off + i, L)]
                q1 = plsc.load_gather(table, [cidx])
                q2 = plsc.load_gather(table, [nidx])
                x = jnp.clip((d - R_ON) * SCALE, 0.0, PI) - HALF_PI
                x2 = x * x
                sinx_c = x * (C0 + x2 * (C3 + x2 * (C5 + x2 * C7)))
                pair = (C0 - sinx_c) * q1 * q2 / d
                plsc.addupdate_scatter(acc, [cidx], pair)

        tcopy = pltpu.async_copy(charges_hbm, table.at[pl.ds(0, n)], tsem)
        issue(0, 0, sem0)

        zero16 = jnp.zeros((L,), jnp.float32)

        @plsc.parallel_loop(0, n_pad, L, unroll=8)
        def _(i):
            acc[pl.ds(i, L)] = zero16

        tcopy.wait()

        @pl.loop(0, nchunk, step=2)
        def _(j):
            @pl.when(j + 1 < nchunk)
            def _():
                issue(j + 1, 1, sem1)

            drain(0, sem0)
            compute(0)

            @pl.when(j + 2 < nchunk)
            def _():
                issue(j + 2, 0, sem0)

            @pl.when(j + 1 < nchunk)
            def _():
                drain(1, sem1)
                compute(1)

        pltpu.sync_copy(acc, out_hbm.at[pl.ds(wid * n_pad, n_pad)])

    return kern


def _tc_sum(partials_flat, n, n_pad):
    def body(p_ref, o_ref):
        acc = p_ref[pl.ds(0, n_pad)]
        for w in range(1, NW):
            acc = acc + p_ref[pl.ds(w * n_pad, n_pad)]
        o_ref[...] = acc[:n]

    return pl.pallas_call(
        body,
        out_shape=jax.ShapeDtypeStruct((n,), jnp.float32),
    )(partials_flat)


def kernel(long_edge_index, long_edge_length, atomic_charges):
    n = atomic_charges.shape[0]
    e = long_edge_length.shape[0]

    n_pad = ((n + 1 + 255) // 256) * 256
    epb = NW * CH
    e_pad = ((e + epb - 1) // epb) * epb

    length = long_edge_length.astype(jnp.float32)
    idx2 = long_edge_index.astype(jnp.int32)
    charges = atomic_charges.astype(jnp.float32)
    if e_pad != e:
        idx2 = jnp.pad(idx2, ((0, 0), (0, e_pad - e)), constant_values=n)
        length = jnp.pad(length, (0, e_pad - e), constant_values=1.0)
        charges = jnp.pad(charges, (0, n_pad - n))
    idx2_flat = idx2.reshape(2 * e_pad)

    partials = _sc_coulomb(charges.shape[0], n_pad, e_pad)(
        idx2_flat, length, charges)
    return _tc_sum(partials, n, n_pad)
